# Initial kernel scaffold; baseline (speedup 1.0000x reference)
import jax
import jax.numpy as jnp
from jax.experimental import pallas as pl


def kernel(x, edge_index, edge_weight, Wb, bb, W1, b1, W2, b2):
    raise NotImplementedError("write your pallas kernel here")


# speedup vs baseline: 2.9284x; 2.9284x over previous
import functools

import jax
import jax.numpy as jnp
from jax import lax
from jax.experimental import pallas as pl
from jax.experimental.pallas import tpu as pltpu
from jax.experimental.pallas import tpu_sc as plsc

NC = 2
NS = 16
NW = NC * NS
B = 80
L = 16


@functools.cache
def _make_agg(n, e, c):
    npad = ((n + NS * 8 - 1) // (NS * 8)) * (NS * 8)
    assert e % (NW * B) == 0 and c % L == 0
    epw = e // NW
    nchunks = epw // B
    rpt = npad // NS
    nq = c // L
    mesh = plsc.VectorSubcoreMesh(core_axis_name="c", subcore_axis_name="s")

    def body(sup_hbm, src_hbm, dst_hbm, ewb_hbm, zero_hbm, out_hbm,
             src_v, dst_v, ewb_v, rows_v, acc_sh, sem):
        ci = lax.axis_index("c")
        si = lax.axis_index("s")
        wid = si * NC + ci

        pltpu.sync_copy(zero_hbm.at[pl.ds(si * rpt, rpt)],
                        acc_sh.at[pl.ds(si * rpt, rpt)])
        plsc.subcore_barrier()

        def chunk(k, _):
            eb = wid * epw + k * B
            pltpu.sync_copy(src_hbm.at[pl.ds(eb, B)], src_v)
            pltpu.sync_copy(dst_hbm.at[pl.ds(eb, B)], dst_v)
            pltpu.sync_copy(ewb_hbm.at[pl.ds(eb, B)], ewb_v)
            pltpu.async_copy(sup_hbm.at[src_v], rows_v, sem).wait()

            def srow(b, _):
                w = ewb_v[b, pl.ds(0, L)]
                for q in range(nq):
                    rows_v[b, pl.ds(q * L, L)] = rows_v[b, pl.ds(q * L, L)] * w
                return 0

            lax.fori_loop(0, B, srow, 0)
            pltpu.sync_copy(rows_v, acc_sh.at[dst_v], add=True)
            return 0

        lax.fori_loop(0, nchunks, chunk, 0)
        plsc.subcore_barrier()
        pltpu.sync_copy(acc_sh.at[pl.ds(si * rpt, rpt)],
                        out_hbm.at[ci, pl.ds(si * rpt, rpt)])

    return pl.kernel(
        body,
        out_type=jax.ShapeDtypeStruct((NC, npad, c), jnp.float32),
        mesh=mesh,
        compiler_params=pltpu.CompilerParams(use_tc_tiling_on_sc=False),
        scratch_types=[
            pltpu.VMEM((B,), jnp.int32),
            pltpu.VMEM((B,), jnp.int32),
            pltpu.VMEM((B, L), jnp.float32),
            pltpu.VMEM((B, c), jnp.float32),
            pltpu.VMEM_SHARED((npad, c), jnp.float32),
            pltpu.SemaphoreType.DMA,
        ],
    )


def _mm_body(x_ref, w_ref, o_ref):
    o_ref[...] = jnp.dot(x_ref[...], w_ref[...],
                         preferred_element_type=jnp.float32)


def _comb1_body(p_ref, b_ref, w_ref, h_ref, s_ref):
    h = p_ref[0] + p_ref[1] + b_ref[...]
    h_ref[...] = h
    s_ref[...] = jnp.dot(h, w_ref[...], preferred_element_type=jnp.float32)


def _comb2_body(p_ref, b_ref, w_ref, s_ref):
    h = jnp.maximum(p_ref[0] + p_ref[1] + b_ref[...], 0.0)
    s_ref[...] = jnp.dot(h, w_ref[...], preferred_element_type=jnp.float32)


def _final_body(p_ref, b_ref, lg_ref, o_ref):
    t = p_ref[0] + p_ref[1] + b_ref[...]
    o_ref[...] = lg_ref[...] * jnp.log(1.1 + jnp.exp(t))


def kernel(x, edge_index, edge_weight, Wb, bb, W1, b1, W2, b2):
    n, _ = x.shape
    c = Wb.shape[1]
    e = edge_weight.shape[0]
    src = edge_index[0]
    dst = edge_index[1]
    npad = ((n + NS * 8 - 1) // (NS * 8)) * (NS * 8)
    zeros = jnp.zeros((npad, c), jnp.float32)
    agg = _make_agg(n, e, c)
    f32 = jnp.float32

    ewb = jnp.broadcast_to(edge_weight[:, None], (e, L))
    sup_b = pl.pallas_call(
        _mm_body, out_shape=jax.ShapeDtypeStruct((n, c), f32))(x, Wb)
    p_b = agg(sup_b, src, dst, ewb, zeros)[:, :n]
    logits, sup1 = pl.pallas_call(
        _comb1_body,
        out_shape=(jax.ShapeDtypeStruct((n, c), f32),
                   jax.ShapeDtypeStruct((n, c), f32)),
    )(p_b, bb.reshape(1, c), W1)
    p1 = agg(sup1, src, dst, ewb, zeros)[:, :n]
    sup2 = pl.pallas_call(
        _comb2_body, out_shape=jax.ShapeDtypeStruct((n, c), f32),
    )(p1, b1.reshape(1, c), W2)
    p2 = agg(sup2, src, dst, ewb, zeros)[:, :n]
    out = pl.pallas_call(
        _final_body, out_shape=jax.ShapeDtypeStruct((n, c), f32),
    )(p2, b2.reshape(1, c), logits)
    return out


# speedup vs baseline: 4.8252x; 1.6477x over previous
import functools

import jax
import jax.numpy as jnp
from jax import lax
from jax.experimental import pallas as pl
from jax.experimental.pallas import tpu as pltpu
from jax.experimental.pallas import tpu_sc as plsc

NC = 2
NS = 16
NW = NC * NS
B = 80
L = 16


@functools.cache
def _make_agg(n, e, c):
    npad = ((n + NS * 8 - 1) // (NS * 8)) * (NS * 8)
    assert e % (NW * B) == 0 and c % L == 0
    epw = e // NW
    nchunks = epw // B
    rpt = npad // NS
    nq = c // L
    mesh = plsc.VectorSubcoreMesh(core_axis_name="c", subcore_axis_name="s")

    def body(sup_hbm, src_hbm, dst_hbm, ewb_hbm, zero_hbm, out_hbm,
             src_v0, src_v1, dst_v0, dst_v1, ewb_v0, ewb_v1,
             rows_v0, rows_v1, acc_sh, rsem0, rsem1, isem):
        ci = lax.axis_index("c")
        si = lax.axis_index("s")
        wid = si * NC + ci
        base = wid * epw
        src_v = (src_v0, src_v1)
        dst_v = (dst_v0, dst_v1)
        ewb_v = (ewb_v0, ewb_v1)
        rows_v = (rows_v0, rows_v1)
        rsem = (rsem0, rsem1)

        pltpu.sync_copy(zero_hbm.at[pl.ds(si * rpt, rpt)],
                        acc_sh.at[pl.ds(si * rpt, rpt)])

        def issue_idx(k, p):
            eb = base + k * B
            pltpu.async_copy(src_hbm.at[pl.ds(eb, B)], src_v[p], isem)
            pltpu.async_copy(dst_hbm.at[pl.ds(eb, B)], dst_v[p], isem)
            pltpu.async_copy(ewb_hbm.at[pl.ds(eb, B)], ewb_v[p], isem)

        def wait_idx(p):
            pltpu.make_async_copy(src_hbm.at[pl.ds(0, B)], src_v[p], isem).wait()
            pltpu.make_async_copy(dst_hbm.at[pl.ds(0, B)], dst_v[p], isem).wait()
            pltpu.make_async_copy(ewb_hbm.at[pl.ds(0, B)], ewb_v[p], isem).wait()

        def scale(p):
            def srow(b, _):
                w = ewb_v[p][b, pl.ds(0, L)]
                for q in range(nq):
                    rows_v[p][b, pl.ds(q * L, L)] = (
                        rows_v[p][b, pl.ds(q * L, L)] * w)
                return 0
            lax.fori_loop(0, B, srow, 0)

        def step(k, p, q):
            wait_idx(q)
            pltpu.async_copy(sup_hbm.at[src_v[q]], rows_v[q], rsem[q])
            pltpu.make_async_copy(sup_hbm.at[src_v[p]], rows_v[p], rsem[p]).wait()
            scale(p)
            pltpu.sync_copy(rows_v[p], acc_sh.at[dst_v[p]], add=True)
            issue_idx(jnp.minimum(k + 2, nchunks - 1), p)

        issue_idx(0, 0)
        wait_idx(0)
        pltpu.async_copy(sup_hbm.at[src_v[0]], rows_v[0], rsem[0])
        issue_idx(1, 1)
        plsc.subcore_barrier()

        def pair(i, _):
            k = 2 * i
            step(k, 0, 1)
            step(k + 1, 1, 0)
            return 0

        lax.fori_loop(0, nchunks // 2, pair, 0)
        if nchunks % 2:
            pe = (nchunks - 1) % 2
            pltpu.make_async_copy(sup_hbm.at[src_v[pe]], rows_v[pe],
                                  rsem[pe]).wait()
            scale(pe)
            pltpu.sync_copy(rows_v[pe], acc_sh.at[dst_v[pe]], add=True)
            wait_idx(1 - pe)

        plsc.subcore_barrier()
        pltpu.sync_copy(acc_sh.at[pl.ds(si * rpt, rpt)],
                        out_hbm.at[ci, pl.ds(si * rpt, rpt)])

    return pl.kernel(
        body,
        out_type=jax.ShapeDtypeStruct((NC, npad, c), jnp.float32),
        mesh=mesh,
        compiler_params=pltpu.CompilerParams(use_tc_tiling_on_sc=False),
        scratch_types=[
            pltpu.VMEM((B,), jnp.int32), pltpu.VMEM((B,), jnp.int32),
            pltpu.VMEM((B,), jnp.int32), pltpu.VMEM((B,), jnp.int32),
            pltpu.VMEM((B, L), jnp.float32), pltpu.VMEM((B, L), jnp.float32),
            pltpu.VMEM((B, c), jnp.float32), pltpu.VMEM((B, c), jnp.float32),
            pltpu.VMEM_SHARED((npad, c), jnp.float32),
            pltpu.SemaphoreType.DMA, pltpu.SemaphoreType.DMA,
            pltpu.SemaphoreType.DMA,
        ],
    )


def _mm_body(x_ref, w_ref, o_ref):
    o_ref[...] = jnp.dot(x_ref[...], w_ref[...],
                         preferred_element_type=jnp.float32)


def _comb1_body(p_ref, b_ref, w_ref, h_ref, s_ref):
    h = p_ref[0] + p_ref[1] + b_ref[...]
    h_ref[...] = h
    s_ref[...] = jnp.dot(h, w_ref[...], preferred_element_type=jnp.float32)


def _comb2_body(p_ref, b_ref, w_ref, s_ref):
    h = jnp.maximum(p_ref[0] + p_ref[1] + b_ref[...], 0.0)
    s_ref[...] = jnp.dot(h, w_ref[...], preferred_element_type=jnp.float32)


def _final_body(p_ref, b_ref, lg_ref, o_ref):
    t = p_ref[0] + p_ref[1] + b_ref[...]
    o_ref[...] = lg_ref[...] * jnp.log(1.1 + jnp.exp(t))


def kernel(x, edge_index, edge_weight, Wb, bb, W1, b1, W2, b2):
    n, _ = x.shape
    c = Wb.shape[1]
    e = edge_weight.shape[0]
    src = edge_index[0]
    dst = edge_index[1]
    npad = ((n + NS * 8 - 1) // (NS * 8)) * (NS * 8)
    zeros = jnp.zeros((npad, c), jnp.float32)
    agg = _make_agg(n, e, c)
    f32 = jnp.float32

    ewb = jnp.broadcast_to(edge_weight[:, None], (e, L))
    sup_b = pl.pallas_call(
        _mm_body, out_shape=jax.ShapeDtypeStruct((n, c), f32))(x, Wb)
    p_b = agg(sup_b, src, dst, ewb, zeros)[:, :n]
    logits, sup1 = pl.pallas_call(
        _comb1_body,
        out_shape=(jax.ShapeDtypeStruct((n, c), f32),
                   jax.ShapeDtypeStruct((n, c), f32)),
    )(p_b, bb.reshape(1, c), W1)
    p1 = agg(sup1, src, dst, ewb, zeros)[:, :n]
    sup2 = pl.pallas_call(
        _comb2_body, out_shape=jax.ShapeDtypeStruct((n, c), f32),
    )(p1, b1.reshape(1, c), W2)
    p2 = agg(sup2, src, dst, ewb, zeros)[:, :n]
    out = pl.pallas_call(
        _final_body, out_shape=jax.ShapeDtypeStruct((n, c), f32),
    )(p2, b2.reshape(1, c), logits)
    return out
